# baseline (speedup 1.0000x reference)
import functools

import jax
import jax.numpy as jnp
from jax import lax
from jax.experimental import pallas as pl
from jax.experimental.pallas import tpu as pltpu
from jax.experimental.pallas import tpu_sc as plsc

N = 10000
D_FEAT = 256
EMBED = 128
K = 32
M = N * K + N
M_PAD = 331776

NPAD = 10240
KNN_B = 200
MLP_B = 2048
OUT_B = 4096

N_WORKERS = 32
CPW = M_PAD // N_WORKERS
CHUNK = 64
N_CHUNKS = CPW // CHUNK


NCH = 128
CSZ = NPAD // NCH
TOPC = 8


def _knn_body(pos_ref, posal3_ref, idx_ref):
    blk = pl.program_id(0)
    laneid = lax.broadcasted_iota(jnp.int32, (8, NCH), 1)
    kid = lax.broadcasted_iota(jnp.int32, (8, K), 1)

    def rb_body(rb, carry):
        r0 = rb * 8
        rpx = pos_ref[pl.ds(r0, 8), 0:1]
        rpy = pos_ref[pl.ds(r0, 8), 1:2]
        rsq = rpx * rpx + rpy * rpy
        bpx = rpx.astype(jnp.bfloat16).astype(jnp.float32)
        bpy = rpy.astype(jnp.bfloat16).astype(jnp.float32)
        rowid = (lax.broadcasted_iota(jnp.int32, (8, 1), 0)
                 + (blk * KNN_B + r0))

        def s_body(s, lists):
            Ls, Is = lists
            cpx = posal3_ref[0:1, pl.ds(s, 1), :].reshape(1, NCH)
            cpy = posal3_ref[1:2, pl.ds(s, 1), :].reshape(1, NCH)
            csq = cpx * cpx + cpy * cpy
            bcx = cpx.astype(jnp.bfloat16).astype(jnp.float32)
            bcy = cpy.astype(jnp.bfloat16).astype(jnp.float32)
            v = rsq + csq - 2.0 * (bpx * bcx + bpy * bcy)
            col = laneid + s * NCH
            v = jnp.where(col == rowid, 1e30, v)
            iv = col
            nLs, nIs = [], []
            for i in range(TOPC):
                sw = v < Ls[i]
                nLs.append(jnp.where(sw, v, Ls[i]))
                nIs.append(jnp.where(sw, iv, Is[i]))
                v2 = jnp.where(sw, Ls[i], v)
                iv = jnp.where(sw, Is[i], iv)
                v = v2
            return tuple(nLs), tuple(nIs)

        Ls0 = tuple(jnp.full((8, NCH), 1e30, jnp.float32) for _ in range(TOPC))
        Is0 = tuple(jnp.full((8, NCH), NPAD, jnp.int32) for _ in range(TOPC))
        Ls, Is = lax.fori_loop(0, CSZ, s_body, (Ls0, Is0))

        def x_body(j, carry2):
            Ls2, acc = carry2
            m8 = Ls2[0]
            for i in range(1, TOPC):
                m8 = jnp.minimum(m8, Ls2[i])
            rowm = jnp.min(m8, axis=1, keepdims=True)
            a8 = jnp.full((8, NCH), NPAD, jnp.int32)
            for i in range(TOPC):
                a8 = jnp.minimum(a8, jnp.where(Ls2[i] <= rowm, Is[i], NPAD))
            am = jnp.min(a8, axis=1, keepdims=True)
            acc = jnp.where(kid == j, am, acc)
            Ls2 = tuple(jnp.where(Is[i] == am, 1e30, Ls2[i])
                        for i in range(TOPC))
            return Ls2, acc

        _, acc = lax.fori_loop(0, K, x_body,
                               (Ls, jnp.zeros((8, K), jnp.int32)))
        idx_ref[pl.ds(r0, 8), :] = acc
        return carry

    lax.fori_loop(0, KNN_B // 8, rb_body, 0)


def _knn(pos8, posal3):
    return pl.pallas_call(
        _knn_body,
        grid=(N // KNN_B,),
        in_specs=[
            pl.BlockSpec((KNN_B, 8), lambda i: (i, 0)),
            pl.BlockSpec((2, CSZ, NCH), lambda i: (0, 0, 0)),
        ],
        out_specs=pl.BlockSpec((KNN_B, K), lambda i: (i, 0)),
        out_shape=jax.ShapeDtypeStruct((N, K), jnp.int32),
    )(pos8, posal3)


def _norm_body(x_ref, o_ref):
    ssq = jnp.sum(x_ref[...] * x_ref[...], axis=1, keepdims=True)
    inv = 1.0 / jnp.maximum(jnp.sqrt(ssq), 1e-8)
    o_ref[...] = jnp.broadcast_to(inv, o_ref.shape)


def _norms(x):
    return pl.pallas_call(
        _norm_body,
        grid=(N // KNN_B,),
        in_specs=[pl.BlockSpec((KNN_B, D_FEAT), lambda i: (i, 0))],
        out_specs=pl.BlockSpec((KNN_B, 8), lambda i: (i, 0)),
        out_shape=jax.ShapeDtypeStruct((N, 8), jnp.float32),
    )(x)


def _sc_edge_body(x_hbm, cols_hbm, src_hbm, tgt_hbm, out_hbm,
                  sidx, tidx, xs, xt, colv, ob, sem):
    wid = lax.axis_index("s") * 2 + lax.axis_index("c")
    base0 = wid * CPW
    lane = lax.iota(jnp.int32, 16)

    for c in range(9):
        pltpu.sync_copy(cols_hbm[c], colv[c])

    def chunk_body(ch, carry):
        base = base0 + ch * CHUNK
        pltpu.sync_copy(src_hbm.at[pl.ds(base, CHUNK)], sidx)
        pltpu.sync_copy(tgt_hbm.at[pl.ds(base, CHUNK)], tidx)
        pltpu.async_copy(x_hbm.at[sidx], xs, sem).wait()
        pltpu.async_copy(x_hbm.at[tidx], xt, sem).wait()

        def grp(g, carry2):
            rows = g * 16 + lane
            si = sidx[pl.ds(g * 16, 16)]
            ti = tidx[pl.ds(g * 16, 16)]

            def scol(c):
                return plsc.load_gather(colv[c], [si])

            def tcol(c):
                return plsc.load_gather(colv[c], [ti])

            s0, s1, s2, s3 = scol(0), scol(1), scol(2), scol(3)
            s4, s5, s6, s7, s8 = scol(4), scol(5), scol(6), scol(7), scol(8)
            t0, t1, t2, t3 = tcol(0), tcol(1), tcol(2), tcol(3)
            t4, t5, t6, t7, t8 = tcol(4), tcol(5), tcol(6), tcol(7), tcol(8)

            wsum = s5 + t5
            f1 = 2.0 * (s6 - t6) / wsum
            f2 = 2.0 * (s7 - t7) / wsum
            r3 = s5 / t5
            r4 = s4 / t4
            ix1 = jnp.maximum(s0, t0)
            iy1 = jnp.maximum(s1, t1)
            ix2 = jnp.minimum(s2, t2)
            iy2 = jnp.minimum(s3, t3)
            inter = jnp.maximum(ix2 - ix1, 0.0) * jnp.maximum(iy2 - iy1, 0.0)
            sa = s4 * s5
            ta = t4 * t5
            union = sa + ta - inter
            invu = 1.0 / (union + 1e-8)
            ixc = (ix1 + ix2) * 0.5
            iyc = (iy1 + iy2) * 0.5
            uxc = (s6 * sa + t6 * ta) * invu
            uyc = (s7 * sa + t7 * ta) * invu
            dist2 = (ixc - uxc) * (ixc - uxc) + (iyc - uyc) * (iyc - uyc)
            f5 = 1.0 - (inter * invu - dist2 * invu)

            dots = jnp.zeros((16,), jnp.float32)
            for e in range(16):
                row = g * 16 + e
                acc = jnp.zeros((16,), jnp.float32)
                for kk in range(D_FEAT // 16):
                    a = xs[row, pl.ds(kk * 16, 16)]
                    b = xt[row, pl.ds(kk * 16, 16)]
                    acc = acc + a * b
                d = jnp.sum(acc)
                dots = jnp.where(lane == e, d, dots)
            f6 = 1.0 - dots * s8 * t8

            obase = rows * 8
            plsc.store_scatter(ob, [obase + 0], f1)
            plsc.store_scatter(ob, [obase + 1], f2)
            plsc.store_scatter(ob, [obase + 2], r3)
            plsc.store_scatter(ob, [obase + 3], r4)
            plsc.store_scatter(ob, [obase + 4], f5)
            plsc.store_scatter(ob, [obase + 5], f6)
            zero = jnp.zeros((16,), jnp.float32)
            plsc.store_scatter(ob, [obase + 6], zero)
            plsc.store_scatter(ob, [obase + 7], zero)
            return carry2

        lax.fori_loop(0, CHUNK // 16, grp, 0)
        pltpu.sync_copy(ob, out_hbm.at[pl.ds(base * 8, CHUNK * 8)])
        return carry

    lax.fori_loop(0, N_CHUNKS, chunk_body, 0)


def _sc_edge_features(x, info_cols, src, tgt):
    mesh = plsc.VectorSubcoreMesh(core_axis_name="c", subcore_axis_name="s")
    kern = functools.partial(
        pl.kernel,
        mesh=mesh,
        compiler_params=pltpu.CompilerParams(needs_layout_passes=False),
        out_type=jax.ShapeDtypeStruct((M_PAD * 8,), jnp.float32),
        scratch_types=[
            pltpu.VMEM((CHUNK,), jnp.int32),
            pltpu.VMEM((CHUNK,), jnp.int32),
            pltpu.VMEM((CHUNK, D_FEAT), jnp.float32),
            pltpu.VMEM((CHUNK, D_FEAT), jnp.float32),
            [pltpu.VMEM((N,), jnp.float32) for _ in range(9)],
            pltpu.VMEM((CHUNK * 8,), jnp.float32),
            pltpu.SemaphoreType.DMA,
        ],
    )(_sc_edge_body)
    return kern(x, info_cols, src, tgt)


def _logt(r):
    ci = lax.broadcasted_iota(jnp.int32, r.shape, 1)
    sel = (ci == 2) | (ci == 3)
    lg = jnp.log(jnp.where(sel, jnp.maximum(r, 1e-30), 1.0))
    return jnp.where(sel, lg, r)


def _stats1_body(raw_ref, a_ref, b1_ref, st_ref):
    blk = pl.program_id(0)
    h = jnp.dot(_logt(raw_ref[...]), a_ref[...],
                preferred_element_type=jnp.float32) + b1_ref[0:1, :]
    rowid = lax.broadcasted_iota(jnp.int32, h.shape, 0) + blk * MLP_B
    hm = jnp.where(rowid < M, h, 0.0)
    s = jnp.sum(hm, axis=0, keepdims=True)
    q = jnp.sum(hm * hm, axis=0, keepdims=True)

    @pl.when(blk == 0)
    def _():
        st_ref[...] = jnp.zeros_like(st_ref)

    st_ref[0:1, :] += s
    st_ref[1:2, :] += q


def _stats1(raw, a, b1r):
    return pl.pallas_call(
        _stats1_body,
        grid=(M_PAD // MLP_B,),
        in_specs=[
            pl.BlockSpec((MLP_B, 8), lambda i: (i, 0)),
            pl.BlockSpec((8, EMBED), lambda i: (0, 0)),
            pl.BlockSpec((8, EMBED), lambda i: (0, 0)),
        ],
        out_specs=pl.BlockSpec((8, EMBED), lambda i: (0, 0)),
        out_shape=jax.ShapeDtypeStruct((8, EMBED), jnp.float32),
    )(raw, a, b1r)


def _pass2_body(raw_ref, a1_ref, c1_ref, w2_ref, b2_ref, h2_ref, st_ref):
    blk = pl.program_id(0)
    z = jnp.maximum(
        jnp.dot(_logt(raw_ref[...]), a1_ref[...],
                preferred_element_type=jnp.float32) + c1_ref[0:1, :], 0.0)
    h2 = jnp.dot(z, w2_ref[...], preferred_element_type=jnp.float32) + b2_ref[0:1, :]
    h2_ref[...] = h2
    rowid = lax.broadcasted_iota(jnp.int32, h2.shape, 0) + blk * MLP_B
    hm = jnp.where(rowid < M, h2, 0.0)
    s = jnp.sum(hm, axis=0, keepdims=True)
    q = jnp.sum(hm * hm, axis=0, keepdims=True)

    @pl.when(blk == 0)
    def _():
        st_ref[...] = jnp.zeros_like(st_ref)

    st_ref[0:1, :] += s
    st_ref[1:2, :] += q


def _pass2(raw, a1f, c1r, w2t, b2r):
    return pl.pallas_call(
        _pass2_body,
        grid=(M_PAD // MLP_B,),
        in_specs=[
            pl.BlockSpec((MLP_B, 8), lambda i: (i, 0)),
            pl.BlockSpec((8, EMBED), lambda i: (0, 0)),
            pl.BlockSpec((8, EMBED), lambda i: (0, 0)),
            pl.BlockSpec((EMBED, EMBED), lambda i: (0, 0)),
            pl.BlockSpec((8, EMBED), lambda i: (0, 0)),
        ],
        out_specs=[
            pl.BlockSpec((MLP_B, EMBED), lambda i: (i, 0)),
            pl.BlockSpec((8, EMBED), lambda i: (0, 0)),
        ],
        out_shape=[
            jax.ShapeDtypeStruct((M_PAD, EMBED), jnp.float32),
            jax.ShapeDtypeStruct((8, EMBED), jnp.float32),
        ],
    )(raw, a1f, c1r, w2t, b2r)


def _pass3_body(h2_ref, p_ref, o_ref):
    o_ref[...] = jnp.maximum(h2_ref[...] * p_ref[0:1, :] + p_ref[1:2, :], 0.0)


def _pass3(h2, p):
    return pl.pallas_call(
        _pass3_body,
        grid=(M_PAD // OUT_B,),
        in_specs=[
            pl.BlockSpec((OUT_B, EMBED), lambda i: (i, 0)),
            pl.BlockSpec((8, EMBED), lambda i: (0, 0)),
        ],
        out_specs=pl.BlockSpec((OUT_B, EMBED), lambda i: (i, 0)),
        out_shape=jax.ShapeDtypeStruct((M_PAD, EMBED), jnp.float32),
    )(h2, p)


@jax.jit
def _run(x, location_info, W1, b1, g1, be1, W2, b2, g2, be2):
    pos = location_info[:, -2:]
    pos8 = jnp.pad(pos, ((0, 0), (0, 6)))
    pall = jnp.full((2, NPAD), 1e6, jnp.float32)
    pall = pall.at[0, :N].set(pos[:, 0])
    pall = pall.at[1, :N].set(pos[:, 1])
    posal3 = pall.reshape(2, CSZ, NCH)

    idx = _knn(pos8, posal3)
    invn = _norms(x)

    ar = jnp.arange(N, dtype=jnp.int32)
    src = jnp.concatenate([idx.reshape(-1), ar,
                           jnp.zeros((M_PAD - M,), jnp.int32)])
    tgt = jnp.concatenate([jnp.repeat(ar, K), ar,
                           jnp.zeros((M_PAD - M,), jnp.int32)])
    info_cols = [location_info[:, c] for c in range(8)] + [invn[:, 0]]

    raw = _sc_edge_features(x, info_cols, src, tgt).reshape(M_PAD, 8)

    W1p = jnp.pad(W1, ((0, 0), (0, 2)))
    b1r = jnp.broadcast_to(b1[None, :], (8, EMBED))
    st1 = _stats1(raw, W1p.T, b1r)
    mu1 = st1[0] / M
    var1 = st1[1] / M - mu1 * mu1
    sc1 = g1 / jnp.sqrt(var1 + 1e-5)
    a1f = (W1p * sc1[:, None]).T
    c1 = (b1 - mu1) * sc1 + be1
    c1r = jnp.broadcast_to(c1[None, :], (8, EMBED))
    b2r = jnp.broadcast_to(b2[None, :], (8, EMBED))

    h2, st2 = _pass2(raw, a1f, c1r, W2.T, b2r)
    mu2 = st2[0] / M
    var2 = st2[1] / M - mu2 * mu2
    sc2 = g2 / jnp.sqrt(var2 + 1e-5)
    sh2 = be2 - mu2 * sc2
    p = jnp.concatenate([sc2[None, :], sh2[None, :],
                         jnp.zeros((6, EMBED), jnp.float32)], axis=0)

    out = _pass3(h2, p)
    return out[:M]


def kernel(x, location_info, k, W1, b1, g1, be1, W2, b2, g2, be2):
    return _run(x, location_info, W1, b1, g1, be1, W2, b2, g2, be2)


# speedup vs baseline: 3.3591x; 3.3591x over previous
import functools

import jax
import jax.numpy as jnp
from jax import lax
from jax.experimental import pallas as pl
from jax.experimental.pallas import tpu as pltpu
from jax.experimental.pallas import tpu_sc as plsc

N = 10000
D_FEAT = 256
EMBED = 128
K = 32
M = N * K + N
M_PAD = 331776

NPAD = 10240
KNN_B = 200
MLP_B = 2048
OUT_B = 4096

N_WORKERS = 32
CPW = M_PAD // N_WORKERS
CHUNK = 64
N_CHUNKS = CPW // CHUNK


NCH = 128
CSZ = NPAD // NCH
TOPC = 8


def _knn_body(pos_ref, posall_ref, idx_ref):
    blk = pl.program_id(0)
    B = KNN_B
    rpx = pos_ref[:, 0:1][:, :, None]
    rpy = pos_ref[:, 1:2][:, :, None]
    cpx = posall_ref[0:1, :].reshape(1, CSZ, NCH)
    cpy = posall_ref[1:2, :].reshape(1, CSZ, NCH)
    rsq = rpx * rpx + rpy * rpy
    csq = cpx * cpx + cpy * cpy
    bpx = rpx.astype(jnp.bfloat16).astype(jnp.float32)
    bpy = rpy.astype(jnp.bfloat16).astype(jnp.float32)
    bcx = cpx.astype(jnp.bfloat16).astype(jnp.float32)
    bcy = cpy.astype(jnp.bfloat16).astype(jnp.float32)
    d2 = rsq + csq - 2.0 * (bpx * bcx + bpy * bcy)
    gcol = (lax.broadcasted_iota(jnp.int32, (B, CSZ, NCH), 2)
            + NCH * lax.broadcasted_iota(jnp.int32, (B, CSZ, NCH), 1))
    rowid = lax.broadcasted_iota(jnp.int32, (B, CSZ, NCH), 0) + blk * B
    d2 = jnp.where(gcol == rowid, 1e30, d2)

    miota = lax.broadcasted_iota(jnp.int32, (B, TOPC, NCH), 1)

    def phase_a(j, carry):
        d2c, cv, ci = carry
        m = jnp.min(d2c, axis=1)
        am = jnp.min(jnp.where(d2c <= m[:, None, :], gcol, NPAD), axis=1)
        d2c = jnp.where(gcol == am[:, None, :], 1e30, d2c)
        cv = jnp.where(miota == j, m[:, None, :], cv)
        ci = jnp.where(miota == j, am[:, None, :], ci)
        return d2c, cv, ci

    _, cv, ci = lax.fori_loop(
        0, TOPC, phase_a,
        (d2, jnp.full((B, TOPC, NCH), 1e30, jnp.float32),
         jnp.full((B, TOPC, NCH), NPAD, jnp.int32)))

    kid = lax.broadcasted_iota(jnp.int32, (B, K), 1)

    def phase_b(j, carry):
        cvv, acc = carry
        m = jnp.min(cvv, axis=(1, 2))
        sel = cvv <= m[:, None, None]
        am = jnp.min(jnp.where(sel, ci, NPAD), axis=(1, 2))
        acc = jnp.where(kid == j, am[:, None], acc)
        cvv = jnp.where(ci == am[:, None, None], 1e30, cvv)
        return cvv, acc

    _, idx = lax.fori_loop(0, K, phase_b,
                           (cv, jnp.zeros((B, K), jnp.int32)))
    idx_ref[...] = idx


def _knn(pos8, posall):
    return pl.pallas_call(
        _knn_body,
        grid=(N // KNN_B,),
        in_specs=[
            pl.BlockSpec((KNN_B, 8), lambda i: (i, 0)),
            pl.BlockSpec((8, NPAD), lambda i: (0, 0)),
        ],
        out_specs=pl.BlockSpec((KNN_B, K), lambda i: (i, 0)),
        out_shape=jax.ShapeDtypeStruct((N, K), jnp.int32),
    )(pos8, posall)


def _norm_body(x_ref, o_ref):
    ssq = jnp.sum(x_ref[...] * x_ref[...], axis=1, keepdims=True)
    inv = 1.0 / jnp.maximum(jnp.sqrt(ssq), 1e-8)
    o_ref[...] = jnp.broadcast_to(inv, o_ref.shape)


def _norms(x):
    return pl.pallas_call(
        _norm_body,
        grid=(N // KNN_B,),
        in_specs=[pl.BlockSpec((KNN_B, D_FEAT), lambda i: (i, 0))],
        out_specs=pl.BlockSpec((KNN_B, 8), lambda i: (i, 0)),
        out_shape=jax.ShapeDtypeStruct((N, 8), jnp.float32),
    )(x)


def _sc_edge_body(x_hbm, cols_hbm, src_hbm, tgt_hbm, out_hbm,
                  sidx, tidx, xs, xt, colv, ob, sem, sem2):
    wid = lax.axis_index("s") * 2 + lax.axis_index("c")
    base0 = wid * CPW
    lane = lax.iota(jnp.int32, 16)

    for c in range(9):
        pltpu.sync_copy(cols_hbm[c], colv[c])

    def chunk_body(ch, carry):
        base = base0 + ch * CHUNK
        pltpu.sync_copy(src_hbm.at[pl.ds(base, CHUNK)], sidx)
        pltpu.sync_copy(tgt_hbm.at[pl.ds(base, CHUNK)], tidx)
        cp1 = pltpu.async_copy(x_hbm.at[sidx], xs, sem)
        cp2 = pltpu.async_copy(x_hbm.at[tidx], xt, sem2)
        cp1.wait()
        cp2.wait()

        def grp(g, carry2):
            rows = g * 16 + lane
            si = sidx[pl.ds(g * 16, 16)]
            ti = tidx[pl.ds(g * 16, 16)]

            def scol(c):
                return plsc.load_gather(colv[c], [si])

            def tcol(c):
                return plsc.load_gather(colv[c], [ti])

            s0, s1, s2, s3 = scol(0), scol(1), scol(2), scol(3)
            s4, s5, s6, s7, s8 = scol(4), scol(5), scol(6), scol(7), scol(8)
            t0, t1, t2, t3 = tcol(0), tcol(1), tcol(2), tcol(3)
            t4, t5, t6, t7, t8 = tcol(4), tcol(5), tcol(6), tcol(7), tcol(8)

            wsum = s5 + t5
            f1 = 2.0 * (s6 - t6) / wsum
            f2 = 2.0 * (s7 - t7) / wsum
            r3 = s5 / t5
            r4 = s4 / t4
            ix1 = jnp.maximum(s0, t0)
            iy1 = jnp.maximum(s1, t1)
            ix2 = jnp.minimum(s2, t2)
            iy2 = jnp.minimum(s3, t3)
            inter = jnp.maximum(ix2 - ix1, 0.0) * jnp.maximum(iy2 - iy1, 0.0)
            sa = s4 * s5
            ta = t4 * t5
            union = sa + ta - inter
            invu = 1.0 / (union + 1e-8)
            ixc = (ix1 + ix2) * 0.5
            iyc = (iy1 + iy2) * 0.5
            uxc = (s6 * sa + t6 * ta) * invu
            uyc = (s7 * sa + t7 * ta) * invu
            dist2 = (ixc - uxc) * (ixc - uxc) + (iyc - uyc) * (iyc - uyc)
            f5 = 1.0 - (inter * invu - dist2 * invu)

            dots = jnp.zeros((16,), jnp.float32)
            for e in range(16):
                row = g * 16 + e
                acc = jnp.zeros((16,), jnp.float32)
                for kk in range(D_FEAT // 16):
                    a = xs[row, pl.ds(kk * 16, 16)]
                    b = xt[row, pl.ds(kk * 16, 16)]
                    acc = acc + a * b
                d = jnp.sum(acc)
                dots = jnp.where(lane == e, d, dots)
            f6 = 1.0 - dots * s8 * t8

            obase = rows * 8
            plsc.store_scatter(ob, [obase + 0], f1)
            plsc.store_scatter(ob, [obase + 1], f2)
            plsc.store_scatter(ob, [obase + 2], r3)
            plsc.store_scatter(ob, [obase + 3], r4)
            plsc.store_scatter(ob, [obase + 4], f5)
            plsc.store_scatter(ob, [obase + 5], f6)
            zero = jnp.zeros((16,), jnp.float32)
            plsc.store_scatter(ob, [obase + 6], zero)
            plsc.store_scatter(ob, [obase + 7], zero)
            return carry2

        lax.fori_loop(0, CHUNK // 16, grp, 0)
        pltpu.sync_copy(ob, out_hbm.at[pl.ds(base * 8, CHUNK * 8)])
        return carry

    lax.fori_loop(0, N_CHUNKS, chunk_body, 0)


def _sc_edge_features(x, info_cols, src, tgt):
    mesh = plsc.VectorSubcoreMesh(core_axis_name="c", subcore_axis_name="s")
    kern = functools.partial(
        pl.kernel,
        mesh=mesh,
        compiler_params=pltpu.CompilerParams(needs_layout_passes=False),
        out_type=jax.ShapeDtypeStruct((M_PAD * 8,), jnp.float32),
        scratch_types=[
            pltpu.VMEM((CHUNK,), jnp.int32),
            pltpu.VMEM((CHUNK,), jnp.int32),
            pltpu.VMEM((CHUNK, D_FEAT), jnp.float32),
            pltpu.VMEM((CHUNK, D_FEAT), jnp.float32),
            [pltpu.VMEM((N,), jnp.float32) for _ in range(9)],
            pltpu.VMEM((CHUNK * 8,), jnp.float32),
            pltpu.SemaphoreType.DMA,
            pltpu.SemaphoreType.DMA,
        ],
    )(_sc_edge_body)
    return kern(x, info_cols, src, tgt)


def _logt(r):
    ci = lax.broadcasted_iota(jnp.int32, r.shape, 1)
    sel = (ci == 2) | (ci == 3)
    lg = jnp.log(jnp.where(sel, jnp.maximum(r, 1e-30), 1.0))
    return jnp.where(sel, lg, r)


def _stats1_body(raw_ref, a_ref, b1_ref, st_ref):
    blk = pl.program_id(0)
    h = jnp.dot(_logt(raw_ref[...]), a_ref[...],
                preferred_element_type=jnp.float32) + b1_ref[0:1, :]
    rowid = lax.broadcasted_iota(jnp.int32, h.shape, 0) + blk * MLP_B
    hm = jnp.where(rowid < M, h, 0.0)
    s = jnp.sum(hm, axis=0, keepdims=True)
    q = jnp.sum(hm * hm, axis=0, keepdims=True)

    @pl.when(blk == 0)
    def _():
        st_ref[...] = jnp.zeros_like(st_ref)

    st_ref[0:1, :] += s
    st_ref[1:2, :] += q


def _stats1(raw, a, b1r):
    return pl.pallas_call(
        _stats1_body,
        grid=(M_PAD // MLP_B,),
        in_specs=[
            pl.BlockSpec((MLP_B, 8), lambda i: (i, 0)),
            pl.BlockSpec((8, EMBED), lambda i: (0, 0)),
            pl.BlockSpec((8, EMBED), lambda i: (0, 0)),
        ],
        out_specs=pl.BlockSpec((8, EMBED), lambda i: (0, 0)),
        out_shape=jax.ShapeDtypeStruct((8, EMBED), jnp.float32),
    )(raw, a, b1r)


def _pass2_body(raw_ref, a1_ref, c1_ref, w2_ref, b2_ref, h2_ref, st_ref):
    blk = pl.program_id(0)
    z = jnp.maximum(
        jnp.dot(_logt(raw_ref[...]), a1_ref[...],
                preferred_element_type=jnp.float32) + c1_ref[0:1, :], 0.0)
    h2 = jnp.dot(z, w2_ref[...], preferred_element_type=jnp.float32) + b2_ref[0:1, :]
    h2_ref[...] = h2
    rowid = lax.broadcasted_iota(jnp.int32, h2.shape, 0) + blk * MLP_B
    hm = jnp.where(rowid < M, h2, 0.0)
    s = jnp.sum(hm, axis=0, keepdims=True)
    q = jnp.sum(hm * hm, axis=0, keepdims=True)

    @pl.when(blk == 0)
    def _():
        st_ref[...] = jnp.zeros_like(st_ref)

    st_ref[0:1, :] += s
    st_ref[1:2, :] += q


def _pass2(raw, a1f, c1r, w2t, b2r):
    return pl.pallas_call(
        _pass2_body,
        grid=(M_PAD // MLP_B,),
        in_specs=[
            pl.BlockSpec((MLP_B, 8), lambda i: (i, 0)),
            pl.BlockSpec((8, EMBED), lambda i: (0, 0)),
            pl.BlockSpec((8, EMBED), lambda i: (0, 0)),
            pl.BlockSpec((EMBED, EMBED), lambda i: (0, 0)),
            pl.BlockSpec((8, EMBED), lambda i: (0, 0)),
        ],
        out_specs=[
            pl.BlockSpec((MLP_B, EMBED), lambda i: (i, 0)),
            pl.BlockSpec((8, EMBED), lambda i: (0, 0)),
        ],
        out_shape=[
            jax.ShapeDtypeStruct((M_PAD, EMBED), jnp.float32),
            jax.ShapeDtypeStruct((8, EMBED), jnp.float32),
        ],
    )(raw, a1f, c1r, w2t, b2r)


def _pass3_body(h2_ref, p_ref, o_ref):
    o_ref[...] = jnp.maximum(h2_ref[...] * p_ref[0:1, :] + p_ref[1:2, :], 0.0)


def _pass3(h2, p):
    return pl.pallas_call(
        _pass3_body,
        grid=(M_PAD // OUT_B,),
        in_specs=[
            pl.BlockSpec((OUT_B, EMBED), lambda i: (i, 0)),
            pl.BlockSpec((8, EMBED), lambda i: (0, 0)),
        ],
        out_specs=pl.BlockSpec((OUT_B, EMBED), lambda i: (i, 0)),
        out_shape=jax.ShapeDtypeStruct((M_PAD, EMBED), jnp.float32),
    )(h2, p)


@jax.jit
def _run(x, location_info, W1, b1, g1, be1, W2, b2, g2, be2):
    pos = location_info[:, -2:]
    pos8 = jnp.pad(pos, ((0, 0), (0, 6)))
    posall = jnp.full((8, NPAD), 1e6, jnp.float32)
    posall = posall.at[0, :N].set(pos[:, 0])
    posall = posall.at[1, :N].set(pos[:, 1])

    idx = _knn(pos8, posall)
    invn = _norms(x)

    ar = jnp.arange(N, dtype=jnp.int32)
    src = jnp.concatenate([idx.reshape(-1), ar,
                           jnp.zeros((M_PAD - M,), jnp.int32)])
    tgt = jnp.concatenate([jnp.repeat(ar, K), ar,
                           jnp.zeros((M_PAD - M,), jnp.int32)])
    info_cols = [location_info[:, c] for c in range(8)] + [invn[:, 0]]

    raw = _sc_edge_features(x, info_cols, src, tgt).reshape(M_PAD, 8)

    W1p = jnp.pad(W1, ((0, 0), (0, 2)))
    b1r = jnp.broadcast_to(b1[None, :], (8, EMBED))
    st1 = _stats1(raw, W1p.T, b1r)
    mu1 = st1[0] / M
    var1 = st1[1] / M - mu1 * mu1
    sc1 = g1 / jnp.sqrt(var1 + 1e-5)
    a1f = (W1p * sc1[:, None]).T
    c1 = (b1 - mu1) * sc1 + be1
    c1r = jnp.broadcast_to(c1[None, :], (8, EMBED))
    b2r = jnp.broadcast_to(b2[None, :], (8, EMBED))

    h2, st2 = _pass2(raw, a1f, c1r, W2.T, b2r)
    mu2 = st2[0] / M
    var2 = st2[1] / M - mu2 * mu2
    sc2 = g2 / jnp.sqrt(var2 + 1e-5)
    sh2 = be2 - mu2 * sc2
    p = jnp.concatenate([sc2[None, :], sh2[None, :],
                         jnp.zeros((6, EMBED), jnp.float32)], axis=0)

    out = _pass3(h2, p)
    return out[:M]


def kernel(x, location_info, k, W1, b1, g1, be1, W2, b2, g2, be2):
    return _run(x, location_info, W1, b1, g1, be1, W2, b2, g2, be2)
